# baseline (speedup 1.0000x reference)
import functools

import jax
import jax.numpy as jnp
from jax import lax
from jax.experimental import pallas as pl
from jax.experimental.pallas import tpu as pltpu
from jax.experimental.pallas import tpu_sc as plsc

N_RAYS = 65536
N_COARSE = 64
N_FINE = 128
N_ALL = 192
L = 16
NW = 32
RPW = N_RAYS // NW
C = 64
NCHUNK = RPW // C

_mesh = plsc.VectorSubcoreMesh(core_axis_name="c", subcore_axis_name="s")


@functools.partial(
    pl.kernel,
    out_type=[
        jax.ShapeDtypeStruct((N_RAYS * 3 * N_ALL,), jnp.float32),
        jax.ShapeDtypeStruct((N_RAYS * N_ALL,), jnp.float32),
    ],
    mesh=_mesh,
    compiler_params=pltpu.CompilerParams(needs_layout_passes=False),
    scratch_types=[
        pltpu.VMEM((C * N_COARSE,), jnp.float32),
        pltpu.VMEM((C * N_COARSE,), jnp.float32),
        pltpu.VMEM((C * 8,), jnp.float32),
        pltpu.VMEM((N_COARSE,), jnp.float32),
        pltpu.VMEM((N_COARSE,), jnp.float32),
        pltpu.VMEM((N_FINE,), jnp.float32),
        pltpu.VMEM((C * N_ALL,), jnp.float32),
        pltpu.VMEM((C * 3 * N_ALL,), jnp.float32),
    ],
)
def _sc_sampler(rays_hbm, dists_hbm, w_hbm, pts_out, ad_out,
                dbuf, wbuf, rbuf, cdf_t, mid_t, s_t, ad_st, pts_st):
    wid = lax.axis_index("c") * 16 + lax.axis_index("s")
    iota = lax.broadcasted_iota(jnp.int32, (L,), 0)
    inv127 = jnp.float32(1.0 / (N_FINE - 1))
    eps = jnp.float32(1e-5)
    fzero = jnp.zeros((L,), jnp.float32)
    izero = jnp.zeros((L,), jnp.int32)

    for k in range(4):
        cdf_t[pl.ds(L * k, L)] = fzero

    def chunk_body(g, carry):
        base = wid * RPW + g * C
        pltpu.sync_copy(dists_hbm.at[pl.ds(base * N_COARSE, C * N_COARSE)], dbuf)
        pltpu.sync_copy(w_hbm.at[pl.ds(base * N_COARSE, C * N_COARSE)], wbuf)
        pltpu.sync_copy(rays_hbm.at[pl.ds(base * 8, C * 8)], rbuf)

        def ray_body(i, carry2):
            doff = jnp.full((L,), i * N_COARSE, jnp.int32)
            dv, dsh, wv = [], [], []
            for k in range(4):
                idx0 = iota + (L * k)
                idx1 = jnp.minimum(idx0 + 1, N_COARSE - 1)
                dv.append(plsc.load_gather(dbuf, [doff + idx0]))
                dsh.append(plsc.load_gather(dbuf, [doff + idx1]))
                wv.append(plsc.load_gather(wbuf, [doff + idx1]))
            mv = [0.5 * (dv[k] + dsh[k]) for k in range(4)]
            wp = [wv[k] + eps for k in range(4)]
            wp[3] = jnp.where(iota <= 13, wp[3], fzero)
            sums = [jnp.sum(wp[k]) for k in range(4)]
            tot = sums[0] + sums[1] + sums[2] + sums[3]
            rinv = jnp.float32(1.0) / jnp.full((L,), tot, jnp.float32)
            carry_s = jnp.float32(0.0)
            for k in range(4):
                cs = (plsc.cumsum(wp[k]) + carry_s) * rinv
                idxc = iota + (L * k + 1)
                plsc.store_scatter(cdf_t, [jnp.minimum(idxc, N_COARSE - 1)],
                                   cs, mask=idxc <= 62)
                carry_s = carry_s + sums[k]
                mid_t[pl.ds(L * k, L)] = mv[k]

            rcol = []
            for c in range(6):
                rcol.append(plsc.load_gather(
                    rbuf, [jnp.full((L,), i * 8 + c, jnp.int32)]))
            dx, dy, dz, px, py, pz = rcol

            sv = []
            for j in range(8):
                u = (iota + (L * j)).astype(jnp.float32) * inv127
                lo = izero
                for step in (32, 16, 8, 4, 2, 1):
                    cv = plsc.load_gather(cdf_t, [lo + (step - 1)])
                    lo = jnp.where(cv <= u, lo + step, lo)
                below = lo - 1
                above = jnp.minimum(lo, 62)
                cb = plsc.load_gather(cdf_t, [below])
                ca = plsc.load_gather(cdf_t, [above])
                bb = plsc.load_gather(mid_t, [below])
                ba = plsc.load_gather(mid_t, [above])
                den = ca - cb
                den = jnp.where(den < eps, jnp.float32(1.0), den)
                t = (u - cb) / den
                s = bb + t * (ba - bb)
                s_t[pl.ds(L * j, L)] = s
                sv.append(s)

            aoff = jnp.full((L,), i * N_ALL, jnp.int32)
            poff = jnp.full((L,), i * 3 * N_ALL, jnp.int32)

            for j in range(8):
                s = sv[j]
                lo = izero
                for step in (32, 16, 8, 4, 2, 1):
                    dpr = plsc.load_gather(dbuf, [doff + lo + (step - 1)])
                    lo = jnp.where(dpr <= s, lo + step, lo)
                dlast = plsc.load_gather(dbuf, [doff + (N_COARSE - 1)])
                cnt = jnp.where(dlast <= s, lo + 1, lo)
                pos = iota + (L * j) + cnt
                plsc.store_scatter(ad_st, [aoff + pos], s)
                p3 = poff + pos * 3
                plsc.store_scatter(pts_st, [p3], px + dx * s)
                plsc.store_scatter(pts_st, [p3 + 1], py + dy * s)
                plsc.store_scatter(pts_st, [p3 + 2], pz + dz * s)
            for k in range(4):
                d = dv[k]
                lo = izero
                for step in (64, 32, 16, 8, 4, 2, 1):
                    spr = plsc.load_gather(s_t, [lo + (step - 1)])
                    lo = jnp.where(spr < d, lo + step, lo)
                slast = plsc.load_gather(
                    s_t, [jnp.full((L,), N_FINE - 1, jnp.int32)])
                cnt = jnp.where(slast < d, lo + 1, lo)
                pos = iota + (L * k) + cnt
                plsc.store_scatter(ad_st, [aoff + pos], d)
                p3 = poff + pos * 3
                plsc.store_scatter(pts_st, [p3], px + dx * d)
                plsc.store_scatter(pts_st, [p3 + 1], py + dy * d)
                plsc.store_scatter(pts_st, [p3 + 2], pz + dz * d)
            return carry2

        lax.fori_loop(0, C, ray_body, 0)
        pltpu.sync_copy(ad_st, ad_out.at[pl.ds(base * N_ALL, C * N_ALL)])
        pltpu.sync_copy(pts_st, pts_out.at[pl.ds(base * 3 * N_ALL, C * 3 * N_ALL)])
        return carry

    lax.fori_loop(0, NCHUNK, chunk_body, 0)


def kernel(rays, dists, weights, perturb):
    del perturb
    w = weights[:, :, 0].reshape(N_RAYS * N_COARSE)
    rays8 = jnp.pad(rays, ((0, 0), (0, 2))).reshape(N_RAYS * 8)
    dflat = dists.reshape(N_RAYS * N_COARSE)
    flat, ad = _sc_sampler(rays8, dflat, w)
    return flat.reshape(N_RAYS, N_ALL, 3), ad.reshape(N_RAYS, N_ALL)
